# Initial kernel scaffold; baseline (speedup 1.0000x reference)
import jax
import jax.numpy as jnp
from jax.experimental import pallas as pl


def kernel(x, pe_x, pe_y):
    raise NotImplementedError("write your pallas kernel here")


# speedup vs baseline: 2.6931x; 2.6931x over previous
import jax
import jax.numpy as jnp
from jax.experimental import pallas as pl


def _body(pe_x_ref, pe_y_ref, o_ref):
    px = pe_x_ref[...]
    py = pe_y_ref[...]
    ht = py.shape[0]
    px_b = jnp.broadcast_to(px[None, :, :], (ht, 64, 384)).reshape(ht * 64, 384)
    py_b = jnp.broadcast_to(py[:, None, :], (ht, 64, 384)).reshape(ht * 64, 384)
    o_ref[0, :, pl.ds(0, 384)] = px_b
    o_ref[0, :, pl.ds(384, 384)] = py_b


def kernel(x, pe_x, pe_y):
    bsize, _, h, w = x.shape
    d_model = 2 * pe_x.shape[1]
    ht = 16
    out = pl.pallas_call(
        _body,
        grid=(bsize, h // ht),
        in_specs=[
            pl.BlockSpec((w, pe_x.shape[1]), lambda i, j: (0, 0)),
            pl.BlockSpec((ht, pe_y.shape[1]), lambda i, j: (j, 0)),
        ],
        out_specs=pl.BlockSpec((1, ht * w, d_model), lambda i, j: (i, j, 0)),
        out_shape=jax.ShapeDtypeStruct((bsize, h * w, d_model), jnp.float32),
    )(pe_x, pe_y)
    return out
